# Initial kernel scaffold; baseline (speedup 1.0000x reference)
import jax
import jax.numpy as jnp
from jax.experimental import pallas as pl


def kernel(embs1, embs2, m_embs, edge_index, W_hg, b_hg, bn_gamma, bn_beta, attn_in_w, attn_in_b, attn_out_w, attn_out_b, mha_in_w, mha_in_b, mha_out_w, mha_out_b):
    raise NotImplementedError("write your pallas kernel here")


# speedup vs baseline: 5.3986x; 5.3986x over previous
import functools

import jax
import jax.numpy as jnp
from jax import lax
from jax.experimental import pallas as pl
from jax.experimental.pallas import tpu as pltpu
from jax.experimental.pallas import tpu_sc as plsc

HID = 128
NH = 4
HD = HID // NH
K = 10
N = 4096
Q = 2048
EDGES = 65536
NE = 4096

_INTERP = False


def _heads(a):
    return [a[:, h * HD:(h + 1) * HD] for h in range(NH)]


def _mha_block(qb, kv, in_w, in_b, out_w, out_b):
    wq = in_w[0:HID, :]
    wk = in_w[HID:2 * HID, :]
    wv = in_w[2 * HID:3 * HID, :]
    bq = in_b[:, 0:HID]
    bk = in_b[:, HID:2 * HID]
    bv = in_b[:, 2 * HID:3 * HID]
    q = jnp.dot(qb, wq.T, preferred_element_type=jnp.float32) + bq
    k = jnp.dot(kv, wk.T, preferred_element_type=jnp.float32) + bk
    v = jnp.dot(kv, wv.T, preferred_element_type=jnp.float32) + bv
    scale = 1.0 / jnp.sqrt(jnp.float32(HD))
    outs = []
    for qh, kh, vh in zip(_heads(q), _heads(k), _heads(v)):
        s = jnp.dot(qh, kh.T, preferred_element_type=jnp.float32) * scale
        m = jnp.max(s, axis=1, keepdims=True)
        e = jnp.exp(s - m)
        a = e / jnp.sum(e, axis=1, keepdims=True)
        outs.append(jnp.dot(a, vh, preferred_element_type=jnp.float32))
    o = jnp.concatenate(outs, axis=1)
    return jnp.dot(o, out_w.T, preferred_element_type=jnp.float32) + out_b


def _mha1_body(he0_ref, x_ref, inw_ref, inb_ref, outw_ref, outb_ref, o_ref):
    o_ref[...] = _mha_block(he0_ref[...], x_ref[...], inw_ref[...],
                            inb_ref[...], outw_ref[...], outb_ref[...])


def _knn_body(xb_ref, x_ref, he_ref, whg_ref, xw_ref, *, blk):
    xb = xb_ref[...]
    xall = x_ref[...]
    sqb = jnp.sum(xb * xb, axis=1, keepdims=True)
    sqa = jnp.sum(xall * xall, axis=1)[None, :]
    d2 = sqb + sqa - 2.0 * jnp.dot(xb, xall.T, preferred_element_type=jnp.float32)
    m = jnp.sqrt(jnp.maximum(d2, 0.0))
    cols = lax.broadcasted_iota(jnp.int32, (blk, N), 1)
    acc = jnp.zeros((blk, N), jnp.float32)
    for _ in range(K):
        minv = jnp.min(m, axis=1, keepdims=True)
        cand = jnp.where(m == minv, cols, N)
        minidx = jnp.min(cand, axis=1, keepdims=True)
        sel = cand == minidx
        acc += sel.astype(jnp.float32)
        m = jnp.where(sel, jnp.inf, m)
    inter = jnp.dot(acc, he_ref[...], preferred_element_type=jnp.float32)
    y = xb + inter
    xw_ref[...] = jnp.dot(y, whg_ref[...].T, preferred_element_type=jnp.float32)


def _scale_body(ef_ref, degb_ref, out_ref):
    ef = jnp.sum(ef_ref[...], axis=0)
    d = jnp.sum(degb_ref[...], axis=0)[:, 0:1]
    binv = jnp.where(d > 0, 1.0 / d, 0.0)
    out_ref[...] = ef * binv


def _final_body(mq_ref, out0_ref, degd_ref, bhg_ref, g_ref, b_ref,
                inw_ref, inb_ref, outw_ref, outb_ref, o_ref):
    d = jnp.sum(degd_ref[...], axis=0)[:, 0:1]
    dinv = jnp.where(d > 0, 1.0 / d, 0.0)
    out = dinv * jnp.sum(out0_ref[...], axis=0) + bhg_ref[...]
    mu = jnp.mean(out, axis=0, keepdims=True)
    var = jnp.mean((out - mu) ** 2, axis=0, keepdims=True)
    out = g_ref[...] * (out - mu) / jnp.sqrt(var + 1e-5) + b_ref[...]
    out = jnp.where(out > 0, out, jnp.exp(out) - 1.0)
    o_ref[...] = _mha_block(mq_ref[...], out, inw_ref[...], inb_ref[...],
                            outw_ref[...], outb_ref[...])


def _full(*_):
    return (0, 0)


def _full3(*_):
    return (0, 0, 0)


def _rows(i):
    return (i, 0)


def _tc_mha1(he0, x, in_w, in_b, out_w, out_b):
    blk = 512
    return pl.pallas_call(
        _mha1_body,
        grid=(N // blk,),
        in_specs=[
            pl.BlockSpec((blk, HID), _rows),
            pl.BlockSpec((N, HID), _full),
            pl.BlockSpec((3 * HID, HID), _full),
            pl.BlockSpec((1, 3 * HID), _full),
            pl.BlockSpec((HID, HID), _full),
            pl.BlockSpec((1, HID), _full),
        ],
        out_specs=pl.BlockSpec((blk, HID), _rows),
        out_shape=jax.ShapeDtypeStruct((N, HID), jnp.float32),
        interpret=_INTERP,
    )(he0, x, in_w, in_b, out_w, out_b)


def _tc_knn(x, he, w_hg):
    blk = 256
    return pl.pallas_call(
        functools.partial(_knn_body, blk=blk),
        grid=(N // blk,),
        in_specs=[
            pl.BlockSpec((blk, HID), _rows),
            pl.BlockSpec((N, HID), _full),
            pl.BlockSpec((N, HID), _full),
            pl.BlockSpec((HID, HID), _full),
        ],
        out_specs=pl.BlockSpec((blk, HID), _rows),
        out_shape=jax.ShapeDtypeStruct((N, HID), jnp.float32),
        interpret=_INTERP,
    )(x, x, he, w_hg)


def _tc_scale(ef_raw, degb):
    nc = ef_raw.shape[0]
    return pl.pallas_call(
        _scale_body,
        in_specs=[pl.BlockSpec((nc, NE, HID), _full3),
                  pl.BlockSpec((nc, NE, HID), _full3)],
        out_specs=pl.BlockSpec((NE, HID), _full),
        out_shape=jax.ShapeDtypeStruct((NE, HID), jnp.float32),
        interpret=_INTERP,
    )(ef_raw, degb)


def _tc_final(m_embs, out0, degd, b_hg, gamma, beta, in_w, in_b, out_w, out_b):
    blk = 512
    nc = out0.shape[0]
    return pl.pallas_call(
        _final_body,
        grid=(Q // blk,),
        in_specs=[
            pl.BlockSpec((blk, HID), _rows),
            pl.BlockSpec((nc, N, HID), _full3),
            pl.BlockSpec((nc, N, HID), _full3),
            pl.BlockSpec((1, HID), _full),
            pl.BlockSpec((1, HID), _full),
            pl.BlockSpec((1, HID), _full),
            pl.BlockSpec((3 * HID, HID), _full),
            pl.BlockSpec((1, 3 * HID), _full),
            pl.BlockSpec((HID, HID), _full),
            pl.BlockSpec((1, HID), _full),
        ],
        out_specs=pl.BlockSpec((blk, HID), _rows),
        out_shape=jax.ShapeDtypeStruct((Q, HID), jnp.float32),
        interpret=_INTERP,
    )(m_embs, out0, degd, b_hg, gamma, beta, in_w, in_b, out_w, out_b)


def _make_seg_kernel():
    info = plsc.get_sparse_core_info()
    nc, ns = info.num_cores, info.num_subcores
    per_w = EDGES // (nc * ns)
    ch = 128
    nch = per_w // ch
    mesh = plsc.VectorSubcoreMesh(core_axis_name="c", subcore_axis_name="s")

    out_type = [jax.ShapeDtypeStruct((nc, NE, HID), jnp.float32),
                jax.ShapeDtypeStruct((nc, NE, HID), jnp.float32)]
    scratch = [
        pltpu.VMEM((ch,), jnp.int32),
        pltpu.VMEM((ch,), jnp.int32),
        pltpu.VMEM((ch, HID), jnp.float32),
        pltpu.VMEM((ch, HID), jnp.float32),
        pltpu.VMEM_SHARED((NE, HID), jnp.float32),
        pltpu.VMEM_SHARED((NE, HID), jnp.float32),
        pltpu.SemaphoreType.DMA,
    ]

    def body(table, gidx, sidx, ones_h, zeros_h, out, deg, gidx_v, sidx_v,
             rows_v, ones_v, sh_out, sh_deg, sem):
        cid = lax.axis_index("c")
        sid = lax.axis_index("s")

        @pl.when(sid == 0)
        def _():
            pltpu.sync_copy(zeros_h, sh_out)
            pltpu.sync_copy(zeros_h, sh_deg)

        pltpu.sync_copy(ones_h, ones_v)
        plsc.subcore_barrier()
        base = (cid * ns + sid) * per_w
        for c in range(nch):
            off = base + c * ch
            pltpu.sync_copy(gidx.at[pl.ds(off, ch)], gidx_v)
            pltpu.sync_copy(sidx.at[pl.ds(off, ch)], sidx_v)
            pltpu.async_copy(table.at[gidx_v], rows_v, sem).wait()
            pltpu.sync_copy(rows_v, sh_out.at[sidx_v], add=True)
            pltpu.sync_copy(ones_v, sh_deg.at[sidx_v], add=True)
        plsc.subcore_barrier()
        rpw = NE // ns
        sl = pl.ds(sid * rpw, rpw)
        pltpu.sync_copy(sh_out.at[sl], out.at[cid, sl])
        pltpu.sync_copy(sh_deg.at[sl], deg.at[cid, sl])

    return pl.kernel(body, mesh=mesh, out_type=out_type, scratch_types=scratch)


def _hyper_sc(xw, nodes, edges):
    ones_h = jnp.ones((128, HID), jnp.float32)
    zeros_h = jnp.zeros((NE, HID), jnp.float32)
    seg = _make_seg_kernel()
    ef_raw, degb = seg(xw, nodes, edges, ones_h, zeros_h)
    ef = _tc_scale(ef_raw, degb)
    return seg(ef, edges, nodes, ones_h, zeros_h)


def kernel(embs1, embs2, m_embs, edge_index, W_hg, b_hg, bn_gamma, bn_beta,
           attn_in_w, attn_in_b, attn_out_w, attn_out_b,
           mha_in_w, mha_in_b, mha_out_w, mha_out_b):
    x = jnp.concatenate([embs1, embs2], axis=0)
    he0 = jax.random.normal(jax.random.key(1), (N, HID), dtype=jnp.float32)
    he = _tc_mha1(he0, x, attn_in_w, attn_in_b.reshape(1, -1),
                  attn_out_w, attn_out_b.reshape(1, -1))
    xw = _tc_knn(x, he, W_hg)
    nodes = edge_index[0]
    edges = edge_index[1]
    out0, degd = _hyper_sc(xw, nodes, edges)
    return _tc_final(m_embs, out0, degd, b_hg.reshape(1, -1),
                     bn_gamma.reshape(1, -1), bn_beta.reshape(1, -1),
                     mha_in_w, mha_in_b.reshape(1, -1),
                     mha_out_w, mha_out_b.reshape(1, -1))


# speedup vs baseline: 6.6501x; 1.2318x over previous
import functools

import jax
import jax.numpy as jnp
from jax import lax
from jax.experimental import pallas as pl
from jax.experimental.pallas import tpu as pltpu
from jax.experimental.pallas import tpu_sc as plsc

HID = 128
NH = 4
HD = HID // NH
K = 10
N = 4096
Q = 2048
EDGES = 65536
NE = 4096

_INTERP = False


def _heads(a):
    return [a[:, h * HD:(h + 1) * HD] for h in range(NH)]


def _mha_block(qb, kv, in_w, in_b, out_w, out_b):
    wq = in_w[0:HID, :]
    wk = in_w[HID:2 * HID, :]
    wv = in_w[2 * HID:3 * HID, :]
    bq = in_b[:, 0:HID]
    bk = in_b[:, HID:2 * HID]
    bv = in_b[:, 2 * HID:3 * HID]
    q = jnp.dot(qb, wq.T, preferred_element_type=jnp.float32) + bq
    k = jnp.dot(kv, wk.T, preferred_element_type=jnp.float32) + bk
    v = jnp.dot(kv, wv.T, preferred_element_type=jnp.float32) + bv
    scale = 1.0 / jnp.sqrt(jnp.float32(HD))
    outs = []
    for qh, kh, vh in zip(_heads(q), _heads(k), _heads(v)):
        s = jnp.dot(qh, kh.T, preferred_element_type=jnp.float32) * scale
        m = jnp.max(s, axis=1, keepdims=True)
        e = jnp.exp(s - m)
        a = e / jnp.sum(e, axis=1, keepdims=True)
        outs.append(jnp.dot(a, vh, preferred_element_type=jnp.float32))
    o = jnp.concatenate(outs, axis=1)
    return jnp.dot(o, out_w.T, preferred_element_type=jnp.float32) + out_b


def _mha1_body(he0_ref, x_ref, inw_ref, inb_ref, outw_ref, outb_ref, o_ref):
    o_ref[...] = _mha_block(he0_ref[...], x_ref[...], inw_ref[...],
                            inb_ref[...], outw_ref[...], outb_ref[...])


def _knn_body(xb_ref, x_ref, he_ref, whg_ref, xw_ref, *, blk):
    xb = xb_ref[...]
    xall = x_ref[...]
    sqb = jnp.sum(xb * xb, axis=1, keepdims=True)
    sqa = jnp.sum(xall * xall, axis=1)[None, :]
    m = sqb + sqa - 2.0 * jnp.dot(xb, xall.T, preferred_element_type=jnp.float32)
    acc = jnp.zeros((blk, N), jnp.float32)
    for _ in range(K):
        minv = jnp.min(m, axis=1, keepdims=True)
        sel = m == minv
        acc += sel.astype(jnp.float32)
        m = jnp.where(sel, jnp.inf, m)
    inter = jnp.dot(acc, he_ref[...], preferred_element_type=jnp.float32)
    y = xb + inter
    xw_ref[...] = jnp.dot(y, whg_ref[...].T, preferred_element_type=jnp.float32)


def _scale_body(ef_ref, degb_ref, out_ref):
    ef = jnp.sum(ef_ref[...], axis=0)
    d = jnp.sum(degb_ref[...], axis=0)[:, 0:1]
    binv = jnp.where(d > 0, 1.0 / d, 0.0)
    out_ref[...] = ef * binv


def _final_body(mq_ref, out0_ref, degd_ref, bhg_ref, g_ref, b_ref,
                inw_ref, inb_ref, outw_ref, outb_ref, o_ref):
    d = jnp.sum(degd_ref[...], axis=0)[:, 0:1]
    dinv = jnp.where(d > 0, 1.0 / d, 0.0)
    out = dinv * jnp.sum(out0_ref[...], axis=0) + bhg_ref[...]
    mu = jnp.mean(out, axis=0, keepdims=True)
    var = jnp.mean((out - mu) ** 2, axis=0, keepdims=True)
    out = g_ref[...] * (out - mu) / jnp.sqrt(var + 1e-5) + b_ref[...]
    out = jnp.where(out > 0, out, jnp.exp(out) - 1.0)
    o_ref[...] = _mha_block(mq_ref[...], out, inw_ref[...], inb_ref[...],
                            outw_ref[...], outb_ref[...])


def _full(*_):
    return (0, 0)


def _full3(*_):
    return (0, 0, 0)


def _rows(i):
    return (i, 0)


def _tc_mha1(he0, x, in_w, in_b, out_w, out_b):
    blk = 512
    return pl.pallas_call(
        _mha1_body,
        grid=(N // blk,),
        in_specs=[
            pl.BlockSpec((blk, HID), _rows),
            pl.BlockSpec((N, HID), _full),
            pl.BlockSpec((3 * HID, HID), _full),
            pl.BlockSpec((1, 3 * HID), _full),
            pl.BlockSpec((HID, HID), _full),
            pl.BlockSpec((1, HID), _full),
        ],
        out_specs=pl.BlockSpec((blk, HID), _rows),
        out_shape=jax.ShapeDtypeStruct((N, HID), jnp.float32),
        interpret=_INTERP,
    )(he0, x, in_w, in_b, out_w, out_b)


def _tc_knn(x, he, w_hg):
    blk = 256
    return pl.pallas_call(
        functools.partial(_knn_body, blk=blk),
        grid=(N // blk,),
        in_specs=[
            pl.BlockSpec((blk, HID), _rows),
            pl.BlockSpec((N, HID), _full),
            pl.BlockSpec((N, HID), _full),
            pl.BlockSpec((HID, HID), _full),
        ],
        out_specs=pl.BlockSpec((blk, HID), _rows),
        out_shape=jax.ShapeDtypeStruct((N, HID), jnp.float32),
        interpret=_INTERP,
    )(x, x, he, w_hg)


def _tc_scale(ef_raw, degb):
    nc = ef_raw.shape[0]
    return pl.pallas_call(
        _scale_body,
        in_specs=[pl.BlockSpec((nc, NE, HID), _full3),
                  pl.BlockSpec((nc, NE, HID), _full3)],
        out_specs=pl.BlockSpec((NE, HID), _full),
        out_shape=jax.ShapeDtypeStruct((NE, HID), jnp.float32),
        interpret=_INTERP,
    )(ef_raw, degb)


def _tc_final(m_embs, out0, degd, b_hg, gamma, beta, in_w, in_b, out_w, out_b):
    blk = 512
    nc = out0.shape[0]
    return pl.pallas_call(
        _final_body,
        grid=(Q // blk,),
        in_specs=[
            pl.BlockSpec((blk, HID), _rows),
            pl.BlockSpec((nc, N, HID), _full3),
            pl.BlockSpec((nc, N, HID), _full3),
            pl.BlockSpec((1, HID), _full),
            pl.BlockSpec((1, HID), _full),
            pl.BlockSpec((1, HID), _full),
            pl.BlockSpec((3 * HID, HID), _full),
            pl.BlockSpec((1, 3 * HID), _full),
            pl.BlockSpec((HID, HID), _full),
            pl.BlockSpec((1, HID), _full),
        ],
        out_specs=pl.BlockSpec((blk, HID), _rows),
        out_shape=jax.ShapeDtypeStruct((Q, HID), jnp.float32),
        interpret=_INTERP,
    )(m_embs, out0, degd, b_hg, gamma, beta, in_w, in_b, out_w, out_b)


def _make_seg_kernel():
    info = plsc.get_sparse_core_info()
    nc, ns = info.num_cores, info.num_subcores
    per_w = EDGES // (nc * ns)
    ch = 128
    nch = per_w // ch
    mesh = plsc.VectorSubcoreMesh(core_axis_name="c", subcore_axis_name="s")

    out_type = [jax.ShapeDtypeStruct((nc, NE, HID), jnp.float32),
                jax.ShapeDtypeStruct((nc, NE, HID), jnp.float32)]
    scratch = [
        pltpu.VMEM((ch,), jnp.int32),
        pltpu.VMEM((ch,), jnp.int32),
        pltpu.VMEM((ch, HID), jnp.float32),
        pltpu.VMEM((ch, HID), jnp.float32),
        pltpu.VMEM_SHARED((NE, HID), jnp.float32),
        pltpu.VMEM_SHARED((NE, HID), jnp.float32),
        pltpu.SemaphoreType.DMA,
    ]

    def body(table, gidx, sidx, ones_h, zeros_h, out, deg, gidx_v, sidx_v,
             rows_v, ones_v, sh_out, sh_deg, sem):
        cid = lax.axis_index("c")
        sid = lax.axis_index("s")

        @pl.when(sid == 0)
        def _():
            pltpu.sync_copy(zeros_h, sh_out)
            pltpu.sync_copy(zeros_h, sh_deg)

        pltpu.sync_copy(ones_h, ones_v)
        plsc.subcore_barrier()
        base = (cid * ns + sid) * per_w
        for c in range(nch):
            off = base + c * ch
            pltpu.sync_copy(gidx.at[pl.ds(off, ch)], gidx_v)
            pltpu.sync_copy(sidx.at[pl.ds(off, ch)], sidx_v)
            pltpu.async_copy(table.at[gidx_v], rows_v, sem).wait()
            pltpu.sync_copy(rows_v, sh_out.at[sidx_v], add=True)
            pltpu.sync_copy(ones_v, sh_deg.at[sidx_v], add=True)
        plsc.subcore_barrier()
        rpw = NE // ns
        sl = pl.ds(sid * rpw, rpw)
        pltpu.sync_copy(sh_out.at[sl], out.at[cid, sl])
        pltpu.sync_copy(sh_deg.at[sl], deg.at[cid, sl])

    return pl.kernel(body, mesh=mesh, out_type=out_type, scratch_types=scratch)


def _hyper_sc(xw, nodes, edges):
    ones_h = jnp.ones((128, HID), jnp.float32)
    zeros_h = jnp.zeros((NE, HID), jnp.float32)
    seg = _make_seg_kernel()
    ef_raw, degb = seg(xw, nodes, edges, ones_h, zeros_h)
    ef = _tc_scale(ef_raw, degb)
    return seg(ef, edges, nodes, ones_h, zeros_h)


def kernel(embs1, embs2, m_embs, edge_index, W_hg, b_hg, bn_gamma, bn_beta,
           attn_in_w, attn_in_b, attn_out_w, attn_out_b,
           mha_in_w, mha_in_b, mha_out_w, mha_out_b):
    x = jnp.concatenate([embs1, embs2], axis=0)
    he0 = jax.random.normal(jax.random.key(1), (N, HID), dtype=jnp.float32)
    he = _tc_mha1(he0, x, attn_in_w, attn_in_b.reshape(1, -1),
                  attn_out_w, attn_out_b.reshape(1, -1))
    xw = _tc_knn(x, he, W_hg)
    nodes = edge_index[0]
    edges = edge_index[1]
    out0, degd = _hyper_sc(xw, nodes, edges)
    return _tc_final(m_embs, out0, degd, b_hg.reshape(1, -1),
                     bn_gamma.reshape(1, -1), bn_beta.reshape(1, -1),
                     mha_in_w, mha_in_b.reshape(1, -1),
                     mha_out_w, mha_out_b.reshape(1, -1))


# speedup vs baseline: 7.1298x; 1.0721x over previous
import functools

import jax
import jax.numpy as jnp
from jax import lax
from jax.experimental import pallas as pl
from jax.experimental.pallas import tpu as pltpu
from jax.experimental.pallas import tpu_sc as plsc

HID = 128
NH = 4
HD = HID // NH
K = 10
N = 4096
Q = 2048
EDGES = 65536
NE = 4096

_INTERP = False


def _heads(a):
    return [a[:, h * HD:(h + 1) * HD] for h in range(NH)]


def _mha_block(qb, kv, in_w, in_b, out_w, out_b):
    wq = in_w[0:HID, :]
    wk = in_w[HID:2 * HID, :]
    wv = in_w[2 * HID:3 * HID, :]
    bq = in_b[:, 0:HID]
    bk = in_b[:, HID:2 * HID]
    bv = in_b[:, 2 * HID:3 * HID]
    q = jnp.dot(qb, wq.T, preferred_element_type=jnp.float32) + bq
    k = jnp.dot(kv, wk.T, preferred_element_type=jnp.float32) + bk
    v = jnp.dot(kv, wv.T, preferred_element_type=jnp.float32) + bv
    scale = 1.0 / jnp.sqrt(jnp.float32(HD))
    outs = []
    for qh, kh, vh in zip(_heads(q), _heads(k), _heads(v)):
        s = jnp.dot(qh, kh.T, preferred_element_type=jnp.float32) * scale
        m = jnp.max(s, axis=1, keepdims=True)
        e = jnp.exp(s - m)
        a = e / jnp.sum(e, axis=1, keepdims=True)
        outs.append(jnp.dot(a, vh, preferred_element_type=jnp.float32))
    o = jnp.concatenate(outs, axis=1)
    return jnp.dot(o, out_w.T, preferred_element_type=jnp.float32) + out_b


def _mha1_body(he0_ref, x_ref, inw_ref, inb_ref, outw_ref, outb_ref, o_ref):
    o_ref[...] = _mha_block(he0_ref[...], x_ref[...], inw_ref[...],
                            inb_ref[...], outw_ref[...], outb_ref[...])


def _knn_body(xb_ref, x_ref, he_ref, whg_ref, xw_ref, *, blk):
    xb = xb_ref[...]
    xall = x_ref[...]
    sqb = jnp.sum(xb * xb, axis=1, keepdims=True)
    sqa = jnp.sum(xall * xall, axis=1)[None, :]
    m = sqb + sqa - 2.0 * jnp.dot(xb, xall.T, preferred_element_type=jnp.float32)
    acc = jnp.zeros((blk, N), jnp.float32)
    for _ in range(K):
        minv = jnp.min(m, axis=1, keepdims=True)
        sel = m == minv
        acc += sel.astype(jnp.float32)
        m = jnp.where(sel, jnp.inf, m)
    inter = jnp.dot(acc, he_ref[...], preferred_element_type=jnp.float32)
    y = xb + inter
    xw_ref[...] = jnp.dot(y, whg_ref[...].T, preferred_element_type=jnp.float32)


def _scale_body(ef_ref, degb_ref, out_ref):
    ef = jnp.sum(ef_ref[...], axis=0)
    d = jnp.sum(degb_ref[...], axis=0)[:, 0:1]
    binv = jnp.where(d > 0, 1.0 / d, 0.0)
    out_ref[...] = ef * binv


def _final_body(mq_ref, out0_ref, degd_ref, bhg_ref, g_ref, b_ref,
                inw_ref, inb_ref, outw_ref, outb_ref, o_ref):
    d = jnp.sum(degd_ref[...], axis=0)[:, 0:1]
    dinv = jnp.where(d > 0, 1.0 / d, 0.0)
    out = dinv * jnp.sum(out0_ref[...], axis=0) + bhg_ref[...]
    mu = jnp.mean(out, axis=0, keepdims=True)
    var = jnp.mean((out - mu) ** 2, axis=0, keepdims=True)
    out = g_ref[...] * (out - mu) / jnp.sqrt(var + 1e-5) + b_ref[...]
    out = jnp.where(out > 0, out, jnp.exp(out) - 1.0)
    o_ref[...] = _mha_block(mq_ref[...], out, inw_ref[...], inb_ref[...],
                            outw_ref[...], outb_ref[...])


def _full(*_):
    return (0, 0)


def _full3(*_):
    return (0, 0, 0)


def _rows(i):
    return (i, 0)


def _tc_mha1(he0, x, in_w, in_b, out_w, out_b):
    blk = 512
    return pl.pallas_call(
        _mha1_body,
        grid=(N // blk,),
        in_specs=[
            pl.BlockSpec((blk, HID), _rows),
            pl.BlockSpec((N, HID), _full),
            pl.BlockSpec((3 * HID, HID), _full),
            pl.BlockSpec((1, 3 * HID), _full),
            pl.BlockSpec((HID, HID), _full),
            pl.BlockSpec((1, HID), _full),
        ],
        out_specs=pl.BlockSpec((blk, HID), _rows),
        out_shape=jax.ShapeDtypeStruct((N, HID), jnp.float32),
        interpret=_INTERP,
    )(he0, x, in_w, in_b, out_w, out_b)


def _tc_knn(x, he, w_hg):
    blk = 256
    return pl.pallas_call(
        functools.partial(_knn_body, blk=blk),
        grid=(N // blk,),
        in_specs=[
            pl.BlockSpec((blk, HID), _rows),
            pl.BlockSpec((N, HID), _full),
            pl.BlockSpec((N, HID), _full),
            pl.BlockSpec((HID, HID), _full),
        ],
        out_specs=pl.BlockSpec((blk, HID), _rows),
        out_shape=jax.ShapeDtypeStruct((N, HID), jnp.float32),
        interpret=_INTERP,
    )(x, x, he, w_hg)


def _tc_scale(ef_raw, degb):
    nc = ef_raw.shape[0]
    return pl.pallas_call(
        _scale_body,
        in_specs=[pl.BlockSpec((nc, NE, HID), _full3),
                  pl.BlockSpec((nc, NE, HID), _full3)],
        out_specs=pl.BlockSpec((NE, HID), _full),
        out_shape=jax.ShapeDtypeStruct((NE, HID), jnp.float32),
        interpret=_INTERP,
    )(ef_raw, degb)


def _tc_final(m_embs, out0, degd, b_hg, gamma, beta, in_w, in_b, out_w, out_b):
    blk = 512
    nc = out0.shape[0]
    return pl.pallas_call(
        _final_body,
        grid=(Q // blk,),
        in_specs=[
            pl.BlockSpec((blk, HID), _rows),
            pl.BlockSpec((nc, N, HID), _full3),
            pl.BlockSpec((nc, N, HID), _full3),
            pl.BlockSpec((1, HID), _full),
            pl.BlockSpec((1, HID), _full),
            pl.BlockSpec((1, HID), _full),
            pl.BlockSpec((3 * HID, HID), _full),
            pl.BlockSpec((1, 3 * HID), _full),
            pl.BlockSpec((HID, HID), _full),
            pl.BlockSpec((1, HID), _full),
        ],
        out_specs=pl.BlockSpec((blk, HID), _rows),
        out_shape=jax.ShapeDtypeStruct((Q, HID), jnp.float32),
        interpret=_INTERP,
    )(m_embs, out0, degd, b_hg, gamma, beta, in_w, in_b, out_w, out_b)


def _make_seg_kernel():
    info = plsc.get_sparse_core_info()
    nc, ns = info.num_cores, info.num_subcores
    per_w = EDGES // (nc * ns)
    ch = 128
    nch = per_w // ch
    mesh = plsc.VectorSubcoreMesh(core_axis_name="c", subcore_axis_name="s")

    out_type = [jax.ShapeDtypeStruct((nc, NE, HID), jnp.float32),
                jax.ShapeDtypeStruct((nc, NE, HID), jnp.float32)]
    scratch = [
        pltpu.VMEM((ch,), jnp.int32),
        pltpu.VMEM((ch,), jnp.int32),
        pltpu.VMEM((ch,), jnp.int32),
        pltpu.VMEM((ch,), jnp.int32),
        pltpu.VMEM((ch, HID), jnp.float32),
        pltpu.VMEM((ch, HID), jnp.float32),
        pltpu.VMEM((ch, HID), jnp.float32),
        pltpu.VMEM_SHARED((NE, HID), jnp.float32),
        pltpu.VMEM_SHARED((NE, HID), jnp.float32),
        pltpu.SemaphoreType.DMA,
        pltpu.SemaphoreType.DMA,
    ]

    def body(table, gidx, sidx, ones_h, zeros_h, out, deg, gidx_v0, gidx_v1,
             sidx_v0, sidx_v1, rows_v0, rows_v1, ones_v, sh_out, sh_deg,
             sem0, sem1):
        cid = lax.axis_index("c")
        sid = lax.axis_index("s")
        gidx_v = [gidx_v0, gidx_v1]
        sidx_v = [sidx_v0, sidx_v1]
        rows_v = [rows_v0, rows_v1]
        sems = [sem0, sem1]

        @pl.when(sid == 0)
        def _():
            pltpu.sync_copy(zeros_h, sh_out)
            pltpu.sync_copy(zeros_h, sh_deg)

        pltpu.sync_copy(ones_h, ones_v)
        plsc.subcore_barrier()
        base = (cid * ns + sid) * per_w
        handles = [None, None]
        pltpu.sync_copy(gidx.at[pl.ds(base, ch)], gidx_v[0])
        pltpu.sync_copy(sidx.at[pl.ds(base, ch)], sidx_v[0])
        handles[0] = pltpu.async_copy(table.at[gidx_v[0]], rows_v[0], sems[0])
        for c in range(nch):
            cb = c % 2
            nb = (c + 1) % 2
            if c + 1 < nch:
                off = base + (c + 1) * ch
                pltpu.sync_copy(gidx.at[pl.ds(off, ch)], gidx_v[nb])
                pltpu.sync_copy(sidx.at[pl.ds(off, ch)], sidx_v[nb])
                handles[nb] = pltpu.async_copy(
                    table.at[gidx_v[nb]], rows_v[nb], sems[nb])
            handles[cb].wait()
            pltpu.sync_copy(rows_v[cb], sh_out.at[sidx_v[cb]], add=True)
            pltpu.sync_copy(ones_v, sh_deg.at[sidx_v[cb]], add=True)
        plsc.subcore_barrier()
        rpw = NE // ns
        sl = pl.ds(sid * rpw, rpw)
        pltpu.sync_copy(sh_out.at[sl], out.at[cid, sl])
        pltpu.sync_copy(sh_deg.at[sl], deg.at[cid, sl])

    return pl.kernel(body, mesh=mesh, out_type=out_type, scratch_types=scratch)


def _hyper_sc(xw, nodes, edges):
    ones_h = jnp.ones((128, HID), jnp.float32)
    zeros_h = jnp.zeros((NE, HID), jnp.float32)
    seg = _make_seg_kernel()
    ef_raw, degb = seg(xw, nodes, edges, ones_h, zeros_h)
    ef = _tc_scale(ef_raw, degb)
    return seg(ef, edges, nodes, ones_h, zeros_h)


def kernel(embs1, embs2, m_embs, edge_index, W_hg, b_hg, bn_gamma, bn_beta,
           attn_in_w, attn_in_b, attn_out_w, attn_out_b,
           mha_in_w, mha_in_b, mha_out_w, mha_out_b):
    x = jnp.concatenate([embs1, embs2], axis=0)
    he0 = jax.random.normal(jax.random.key(1), (N, HID), dtype=jnp.float32)
    he = _tc_mha1(he0, x, attn_in_w, attn_in_b.reshape(1, -1),
                  attn_out_w, attn_out_b.reshape(1, -1))
    xw = _tc_knn(x, he, W_hg)
    nodes = edge_index[0]
    edges = edge_index[1]
    out0, degd = _hyper_sc(xw, nodes, edges)
    return _tc_final(m_embs, out0, degd, b_hg.reshape(1, -1),
                     bn_gamma.reshape(1, -1), bn_beta.reshape(1, -1),
                     mha_in_w, mha_in_b.reshape(1, -1),
                     mha_out_w, mha_out_b.reshape(1, -1))


# speedup vs baseline: 7.1357x; 1.0008x over previous
import functools

import jax
import jax.numpy as jnp
from jax import lax
from jax.experimental import pallas as pl
from jax.experimental.pallas import tpu as pltpu
from jax.experimental.pallas import tpu_sc as plsc

HID = 128
NH = 4
HD = HID // NH
K = 10
N = 4096
Q = 2048
EDGES = 65536
NE = 4096


def _heads(a):
    return [a[:, h * HD:(h + 1) * HD] for h in range(NH)]


def _mha_block(qb, kv, in_w, in_b, out_w, out_b):
    wq = in_w[0:HID, :]
    wk = in_w[HID:2 * HID, :]
    wv = in_w[2 * HID:3 * HID, :]
    bq = in_b[:, 0:HID]
    bk = in_b[:, HID:2 * HID]
    bv = in_b[:, 2 * HID:3 * HID]
    q = jnp.dot(qb, wq.T, preferred_element_type=jnp.float32) + bq
    k = jnp.dot(kv, wk.T, preferred_element_type=jnp.float32) + bk
    v = jnp.dot(kv, wv.T, preferred_element_type=jnp.float32) + bv
    scale = 1.0 / jnp.sqrt(jnp.float32(HD))
    outs = []
    for qh, kh, vh in zip(_heads(q), _heads(k), _heads(v)):
        s = jnp.dot(qh, kh.T, preferred_element_type=jnp.float32) * scale
        m = jnp.max(s, axis=1, keepdims=True)
        e = jnp.exp(s - m)
        a = e / jnp.sum(e, axis=1, keepdims=True)
        outs.append(jnp.dot(a, vh, preferred_element_type=jnp.float32))
    o = jnp.concatenate(outs, axis=1)
    return jnp.dot(o, out_w.T, preferred_element_type=jnp.float32) + out_b


def _mha1_body(he0_ref, x_ref, inw_ref, inb_ref, outw_ref, outb_ref, o_ref):
    o_ref[...] = _mha_block(he0_ref[...], x_ref[...], inw_ref[...],
                            inb_ref[...], outw_ref[...], outb_ref[...])


def _knn_body(xb_ref, x_ref, he_ref, whg_ref, xw_ref, *, blk):
    xb = xb_ref[...]
    xall = x_ref[...]
    sqb = jnp.sum(xb * xb, axis=1, keepdims=True)
    sqa = jnp.sum(xall * xall, axis=1)[None, :]
    m = sqb + sqa - 2.0 * jnp.dot(xb, xall.T, preferred_element_type=jnp.float32)
    acc = jnp.zeros((blk, N), jnp.float32)
    for _ in range(K):
        minv = jnp.min(m, axis=1, keepdims=True)
        sel = m == minv
        acc += sel.astype(jnp.float32)
        m = jnp.where(sel, jnp.inf, m)
    inter = jnp.dot(acc, he_ref[...], preferred_element_type=jnp.float32)
    y = xb + inter
    xw_ref[...] = jnp.dot(y, whg_ref[...].T, preferred_element_type=jnp.float32)


def _scale_body(ef_ref, degb_ref, out_ref):
    ef = jnp.sum(ef_ref[...], axis=0)
    d = jnp.sum(degb_ref[...], axis=0)[:, 0:1]
    binv = jnp.where(d > 0, 1.0 / d, 0.0)
    out_ref[...] = ef * binv


def _final_body(mq_ref, out0_ref, degd_ref, bhg_ref, g_ref, b_ref,
                inw_ref, inb_ref, outw_ref, outb_ref, o_ref):
    d = jnp.sum(degd_ref[...], axis=0)[:, 0:1]
    dinv = jnp.where(d > 0, 1.0 / d, 0.0)
    out = dinv * jnp.sum(out0_ref[...], axis=0) + bhg_ref[...]
    mu = jnp.mean(out, axis=0, keepdims=True)
    var = jnp.mean((out - mu) ** 2, axis=0, keepdims=True)
    out = g_ref[...] * (out - mu) / jnp.sqrt(var + 1e-5) + b_ref[...]
    out = jnp.where(out > 0, out, jnp.exp(out) - 1.0)
    o_ref[...] = _mha_block(mq_ref[...], out, inw_ref[...], inb_ref[...],
                            outw_ref[...], outb_ref[...])


def _full(*_):
    return (0, 0)


def _full3(*_):
    return (0, 0, 0)


def _rows(i):
    return (i, 0)


def _tc_mha1(he0, x, in_w, in_b, out_w, out_b):
    blk = 512
    return pl.pallas_call(
        _mha1_body,
        grid=(N // blk,),
        in_specs=[
            pl.BlockSpec((blk, HID), _rows),
            pl.BlockSpec((N, HID), _full),
            pl.BlockSpec((3 * HID, HID), _full),
            pl.BlockSpec((1, 3 * HID), _full),
            pl.BlockSpec((HID, HID), _full),
            pl.BlockSpec((1, HID), _full),
        ],
        out_specs=pl.BlockSpec((blk, HID), _rows),
        out_shape=jax.ShapeDtypeStruct((N, HID), jnp.float32),
    )(he0, x, in_w, in_b, out_w, out_b)


def _tc_knn(x, he, w_hg):
    blk = 256
    return pl.pallas_call(
        functools.partial(_knn_body, blk=blk),
        grid=(N // blk,),
        in_specs=[
            pl.BlockSpec((blk, HID), _rows),
            pl.BlockSpec((N, HID), _full),
            pl.BlockSpec((N, HID), _full),
            pl.BlockSpec((HID, HID), _full),
        ],
        out_specs=pl.BlockSpec((blk, HID), _rows),
        out_shape=jax.ShapeDtypeStruct((N, HID), jnp.float32),
    )(x, x, he, w_hg)


def _tc_scale(ef_raw, degb):
    nc = ef_raw.shape[0]
    return pl.pallas_call(
        _scale_body,
        in_specs=[pl.BlockSpec((nc, NE, HID), _full3),
                  pl.BlockSpec((nc, NE, HID), _full3)],
        out_specs=pl.BlockSpec((NE, HID), _full),
        out_shape=jax.ShapeDtypeStruct((NE, HID), jnp.float32),
    )(ef_raw, degb)


def _tc_final(m_embs, out0, degd, b_hg, gamma, beta, in_w, in_b, out_w, out_b):
    blk = 512
    nc = out0.shape[0]
    return pl.pallas_call(
        _final_body,
        grid=(Q // blk,),
        in_specs=[
            pl.BlockSpec((blk, HID), _rows),
            pl.BlockSpec((nc, N, HID), _full3),
            pl.BlockSpec((nc, N, HID), _full3),
            pl.BlockSpec((1, HID), _full),
            pl.BlockSpec((1, HID), _full),
            pl.BlockSpec((1, HID), _full),
            pl.BlockSpec((3 * HID, HID), _full),
            pl.BlockSpec((1, 3 * HID), _full),
            pl.BlockSpec((HID, HID), _full),
            pl.BlockSpec((1, HID), _full),
        ],
        out_specs=pl.BlockSpec((blk, HID), _rows),
        out_shape=jax.ShapeDtypeStruct((Q, HID), jnp.float32),
    )(m_embs, out0, degd, b_hg, gamma, beta, in_w, in_b, out_w, out_b)


def _make_seg_kernel():
    info = plsc.get_sparse_core_info()
    nc, ns = info.num_cores, info.num_subcores
    per_w = EDGES // (nc * ns)
    ch = 128
    nch = per_w // ch
    mesh = plsc.VectorSubcoreMesh(core_axis_name="c", subcore_axis_name="s")

    out_type = [jax.ShapeDtypeStruct((nc, NE, HID), jnp.float32),
                jax.ShapeDtypeStruct((nc, NE, HID), jnp.float32)]
    scratch = [
        pltpu.VMEM((ch,), jnp.int32),
        pltpu.VMEM((ch,), jnp.int32),
        pltpu.VMEM((ch,), jnp.int32),
        pltpu.VMEM((ch,), jnp.int32),
        pltpu.VMEM((ch, HID), jnp.float32),
        pltpu.VMEM((ch, HID), jnp.float32),
        pltpu.VMEM((ch, HID), jnp.float32),
        pltpu.VMEM_SHARED((NE, HID), jnp.float32),
        pltpu.VMEM_SHARED((NE, HID), jnp.float32),
        pltpu.SemaphoreType.DMA,
        pltpu.SemaphoreType.DMA,
    ]

    def body(table, gidx, sidx, ones_h, zeros_h, out, deg, gidx_v0, gidx_v1,
             sidx_v0, sidx_v1, rows_v0, rows_v1, ones_v, sh_out, sh_deg,
             sem0, sem1):
        cid = lax.axis_index("c")
        sid = lax.axis_index("s")
        gidx_v = [gidx_v0, gidx_v1]
        sidx_v = [sidx_v0, sidx_v1]
        rows_v = [rows_v0, rows_v1]
        sems = [sem0, sem1]

        @pl.when(sid == 0)
        def _():
            pltpu.sync_copy(zeros_h, sh_out)
            pltpu.sync_copy(zeros_h, sh_deg)

        pltpu.sync_copy(ones_h, ones_v)
        plsc.subcore_barrier()
        base = (cid * ns + sid) * per_w
        handles = [None, None]
        pltpu.sync_copy(gidx.at[pl.ds(base, ch)], gidx_v[0])
        pltpu.sync_copy(sidx.at[pl.ds(base, ch)], sidx_v[0])
        handles[0] = pltpu.async_copy(table.at[gidx_v[0]], rows_v[0], sems[0])
        for c in range(nch):
            cb = c % 2
            nb = (c + 1) % 2
            if c + 1 < nch:
                off = base + (c + 1) * ch
                pltpu.sync_copy(gidx.at[pl.ds(off, ch)], gidx_v[nb])
                pltpu.sync_copy(sidx.at[pl.ds(off, ch)], sidx_v[nb])
                handles[nb] = pltpu.async_copy(
                    table.at[gidx_v[nb]], rows_v[nb], sems[nb])
            handles[cb].wait()
            pltpu.sync_copy(rows_v[cb], sh_out.at[sidx_v[cb]], add=True)
            pltpu.sync_copy(ones_v, sh_deg.at[sidx_v[cb]], add=True)
        plsc.subcore_barrier()
        rpw = NE // ns
        sl = pl.ds(sid * rpw, rpw)
        pltpu.sync_copy(sh_out.at[sl], out.at[cid, sl])
        pltpu.sync_copy(sh_deg.at[sl], deg.at[cid, sl])

    return pl.kernel(body, mesh=mesh, out_type=out_type, scratch_types=scratch)


def _hyper_sc(xw, nodes, edges):
    ones_h = jnp.ones((128, HID), jnp.float32)
    zeros_h = jnp.zeros((NE, HID), jnp.float32)
    seg = _make_seg_kernel()
    ef_raw, degb = seg(xw, nodes, edges, ones_h, zeros_h)
    ef = _tc_scale(ef_raw, degb)
    return seg(ef, edges, nodes, ones_h, zeros_h)


def kernel(embs1, embs2, m_embs, edge_index, W_hg, b_hg, bn_gamma, bn_beta,
           attn_in_w, attn_in_b, attn_out_w, attn_out_b,
           mha_in_w, mha_in_b, mha_out_w, mha_out_b):
    x = jnp.concatenate([embs1, embs2], axis=0)
    he0 = jax.random.normal(jax.random.key(1), (N, HID), dtype=jnp.float32)
    he = _tc_mha1(he0, x, attn_in_w, attn_in_b.reshape(1, -1),
                  attn_out_w, attn_out_b.reshape(1, -1))
    xw = _tc_knn(x, he, W_hg)
    nodes = edge_index[0]
    edges = edge_index[1]
    out0, degd = _hyper_sc(xw, nodes, edges)
    return _tc_final(m_embs, out0, degd, b_hg.reshape(1, -1),
                     bn_gamma.reshape(1, -1), bn_beta.reshape(1, -1),
                     mha_in_w, mha_in_b.reshape(1, -1),
                     mha_out_w, mha_out_b.reshape(1, -1))
